# baseline (speedup 1.0000x reference)
import jax
import jax.numpy as jnp
from jax.experimental import pallas as pl
from jax.experimental.pallas import tpu as pltpu

NUM_EXPERTS = 3
D_MODEL = 768
D_FF = 1536
N_TOK = 2048
BLK = 1024


def _body(x_ref, sw_ref, sb_ref, olnw_ref, olnb_ref, mlnw_ref, mlnb_ref,
          W1_hbm, b1_ref, W2_hbm, b2_ref, out_ref, aux_ref,
          W1s_ref, W2s_ref, hs_ref, acc_ref,
          c0_ref, c1_ref, c2_ref, sm_ref, sem1, sem2):
    p = pl.program_id(0)
    i = pl.program_id(1)
    nblk = pl.num_programs(1)
    rows = pl.ds(i * BLK, BLK)

    def w_copies(q, slot):
        return (pltpu.make_async_copy(W1_hbm.at[q], W1s_ref.at[slot], sem1),
                pltpu.make_async_copy(W2_hbm.at[q], W2s_ref.at[slot], sem2))

    @pl.when((i == 0) & (p < NUM_EXPERTS))
    def _():
        c1, c2 = w_copies(p, p % 2)
        c1.start()
        c2.start()

    @pl.when((i == 0) & (p >= 1))
    def _():
        q = p - 1
        slot = q % 2
        c1, c2 = w_copies(q, slot)
        c1.wait()
        c2.wait()

    @pl.when(p == 0)
    def _():
        xb = x_ref[...]
        ones_col = jnp.ones((D_MODEL, 1), jnp.float32)
        swx = jnp.concatenate([sw_ref[...], ones_col], axis=1)
        lg = jnp.dot(xb, swx, preferred_element_type=jnp.float32)
        s2 = jnp.dot(xb * xb, ones_col, preferred_element_type=jnp.float32)
        mu = lg[:, 3:4] * (1.0 / D_MODEL)
        var = s2 * (1.0 / D_MODEL) - mu * mu
        ln = (xb - mu) * jax.lax.rsqrt(var + 1e-6)

        l0 = lg[:, 0:1] + sb_ref[0]
        l1 = lg[:, 1:2] + sb_ref[1]
        l2 = lg[:, 2:3] + sb_ref[2]
        m = jnp.maximum(jnp.maximum(l0, l1), l2)
        e0 = jnp.exp(l0 - m)
        e1 = jnp.exp(l1 - m)
        e2 = jnp.exp(l2 - m)
        z = e0 + e1 + e2
        g0, g1, g2 = e0 / z, e1 / z, e2 / z

        x2 = (g2 <= g0) & (g2 <= g1)
        x1 = jnp.logical_not(x2) & (g1 <= g0)
        x0 = jnp.logical_not(x2) & jnp.logical_not(x1)
        gx = jnp.where(x0, g0, jnp.where(x1, g1, g2))
        inv = 1.0 / ((g0 + g1 + g2) - gx + 1e-6)
        c0_ref[rows, :] = jnp.where(x0, 0.0, g0 * inv)
        c1_ref[rows, :] = jnp.where(x1, 0.0, g1 * inv)
        c2_ref[rows, :] = jnp.where(x2, 0.0, g2 * inv)

        ones = jnp.ones_like(g0)
        G4 = jnp.concatenate([g0, g1, g2, ones], axis=1)
        mixw = jnp.concatenate(
            [mlnw_ref[...], olnw_ref[...].reshape(1, D_MODEL)], axis=0)
        mixb = jnp.concatenate(
            [mlnb_ref[...], olnb_ref[...].reshape(1, D_MODEL)], axis=0)
        mixcat = jnp.concatenate([mixw, mixb], axis=1)
        wb = jnp.dot(G4, mixcat, preferred_element_type=jnp.float32)
        h = ln * wb[:, :D_MODEL] + wb[:, D_MODEL:]
        hs_ref[rows, :] = h

        @pl.when(i == 0)
        def _():
            for k in range(6):
                sm_ref[k] = 0.0

        for k, (g, xe) in enumerate(((g0, x0), (g1, x1), (g2, x2))):
            sm_ref[k] = sm_ref[k] + jnp.sum(g)
            sm_ref[3 + k] = sm_ref[3 + k] + (
                BLK - jnp.sum(xe.astype(jnp.float32)))

    @pl.when(p >= 1)
    def _():
        hs = hs_ref[rows, :]
        slot = (p - 1) % 2
        b1r = jnp.where(p == 1, b1_ref[0:1, :],
                        jnp.where(p == 2, b1_ref[1:2, :], b1_ref[2:3, :]))
        t = jax.lax.dot_general(
            hs, W1s_ref[slot], (((1,), (0,)), ((), ())),
            precision=jax.lax.Precision.DEFAULT,
            preferred_element_type=jnp.float32)
        t = jax.nn.gelu(t + b1r)
        y = jax.lax.dot_general(
            t, W2s_ref[slot], (((1,), (0,)), ((), ())),
            precision=jax.lax.Precision.DEFAULT,
            preferred_element_type=jnp.float32)
        b2r = jnp.where(p == 1, b2_ref[0:1, :],
                        jnp.where(p == 2, b2_ref[1:2, :], b2_ref[2:3, :]))
        y = y + b2r
        c = jnp.where(p == 1, c0_ref[rows, :],
                      jnp.where(p == 2, c1_ref[rows, :], c2_ref[rows, :]))

        @pl.when(p == 1)
        def _():
            acc_ref[rows, :] = hs + c * y

        @pl.when(p == 2)
        def _():
            acc_ref[rows, :] = acc_ref[rows, :] + c * y

        @pl.when(p == NUM_EXPERTS)
        def _():
            out_ref[...] = acc_ref[rows, :] + c * y

            @pl.when(i == nblk - 1)
            def _():
                aux = 0.0
                for k in range(NUM_EXPERTS):
                    aux = aux + (sm_ref[3 + k] / N_TOK) * (sm_ref[k] / N_TOK)
                aux_ref[0, 0] = NUM_EXPERTS * aux


@jax.jit
def kernel(x, scout_W, scout_b, orig_ln_w, orig_ln_b, moe_ln_w, moe_ln_b,
           W1, b1, W2, b2):
    n_tok = x.shape[0]
    grid = (NUM_EXPERTS + 1, n_tok // BLK)
    out, aux = pl.pallas_call(
        _body,
        grid=grid,
        in_specs=[
            pl.BlockSpec((BLK, D_MODEL),
                         lambda p, i: (jnp.where(p == 0, i, 0), 0)),
            pl.BlockSpec((D_MODEL, NUM_EXPERTS), lambda p, i: (0, 0)),
            pl.BlockSpec(memory_space=pltpu.SMEM),
            pl.BlockSpec((D_MODEL,), lambda p, i: (0,)),
            pl.BlockSpec((D_MODEL,), lambda p, i: (0,)),
            pl.BlockSpec((NUM_EXPERTS, D_MODEL), lambda p, i: (0, 0)),
            pl.BlockSpec((NUM_EXPERTS, D_MODEL), lambda p, i: (0, 0)),
            pl.BlockSpec(memory_space=pl.ANY),
            pl.BlockSpec((NUM_EXPERTS, D_FF), lambda p, i: (0, 0)),
            pl.BlockSpec(memory_space=pl.ANY),
            pl.BlockSpec((NUM_EXPERTS, D_MODEL), lambda p, i: (0, 0)),
        ],
        out_specs=[
            pl.BlockSpec((BLK, D_MODEL),
                         lambda p, i: (jnp.where(p == NUM_EXPERTS, i, 0), 0)),
            pl.BlockSpec(memory_space=pltpu.SMEM),
        ],
        out_shape=[
            jax.ShapeDtypeStruct((n_tok, D_MODEL), jnp.float32),
            jax.ShapeDtypeStruct((1, 1), jnp.float32),
        ],
        scratch_shapes=[
            pltpu.VMEM((2, D_MODEL, D_FF), jnp.float32),
            pltpu.VMEM((2, D_FF, D_MODEL), jnp.float32),
            pltpu.VMEM((n_tok, D_MODEL), jnp.float32),
            pltpu.VMEM((n_tok, D_MODEL), jnp.float32),
            pltpu.VMEM((n_tok, 1), jnp.float32),
            pltpu.VMEM((n_tok, 1), jnp.float32),
            pltpu.VMEM((n_tok, 1), jnp.float32),
            pltpu.SMEM((8,), jnp.float32),
            pltpu.SemaphoreType.DMA,
            pltpu.SemaphoreType.DMA,
        ],
        compiler_params=pltpu.CompilerParams(
            dimension_semantics=("arbitrary", "arbitrary")),
    )(
        x, scout_W, scout_b, orig_ln_w, orig_ln_b, moe_ln_w, moe_ln_b,
        W1, b1, W2, b2,
    )
    return out, aux.reshape(())


# speedup vs baseline: 1.0058x; 1.0058x over previous
import jax
import jax.numpy as jnp
from jax.experimental import pallas as pl
from jax.experimental.pallas import tpu as pltpu

NUM_EXPERTS = 3
D_MODEL = 768
D_FF = 1536
N_TOK = 2048
BLK = 1024


def _body(x_ref, sw_ref, sb_ref, olnw_ref, olnb_ref, mlnw_ref, mlnb_ref,
          W1_hbm, b1_ref, W2_hbm, b2_ref, out_ref, aux_ref,
          W1s_ref, W2s_ref, hs_ref, acc_ref,
          c0_ref, c1_ref, c2_ref, sm_ref, sem1, sem2):
    p = pl.program_id(0)
    i = pl.program_id(1)
    nblk = pl.num_programs(1)
    rows = pl.ds(i * BLK, BLK)

    def w_copies(q, slot):
        return (pltpu.make_async_copy(W1_hbm.at[q], W1s_ref.at[slot], sem1),
                pltpu.make_async_copy(W2_hbm.at[q], W2s_ref.at[slot], sem2))

    @pl.when((i == 0) & (p < NUM_EXPERTS))
    def _():
        c1, c2 = w_copies(p, p % 2)
        c1.start()
        c2.start()

    @pl.when((i == 0) & (p >= 1))
    def _():
        q = p - 1
        slot = q % 2
        c1, c2 = w_copies(q, slot)
        c1.wait()
        c2.wait()

    @pl.when(p == 0)
    def _():
        xb = x_ref[...]
        mu = jnp.mean(xb, axis=1, keepdims=True)
        xc = xb - mu
        var = jnp.mean(xc * xc, axis=1, keepdims=True)
        ln = xc * jax.lax.rsqrt(var + 1e-6)

        lg = jnp.dot(xb, sw_ref[...], preferred_element_type=jnp.float32)
        l0 = lg[:, 0:1] + sb_ref[0]
        l1 = lg[:, 1:2] + sb_ref[1]
        l2 = lg[:, 2:3] + sb_ref[2]
        m = jnp.maximum(jnp.maximum(l0, l1), l2)
        e0 = jnp.exp(l0 - m)
        e1 = jnp.exp(l1 - m)
        e2 = jnp.exp(l2 - m)
        z = e0 + e1 + e2
        g0, g1, g2 = e0 / z, e1 / z, e2 / z

        x2 = (g2 <= g0) & (g2 <= g1)
        x1 = jnp.logical_not(x2) & (g1 <= g0)
        x0 = jnp.logical_not(x2) & jnp.logical_not(x1)
        gx = jnp.where(x0, g0, jnp.where(x1, g1, g2))
        inv = 1.0 / ((g0 + g1 + g2) - gx + 1e-6)
        c0_ref[rows, :] = jnp.where(x0, 0.0, g0 * inv)
        c1_ref[rows, :] = jnp.where(x1, 0.0, g1 * inv)
        c2_ref[rows, :] = jnp.where(x2, 0.0, g2 * inv)

        ones = jnp.ones_like(g0)
        G4 = jnp.concatenate([g0, g1, g2, ones], axis=1)
        mixw = jnp.concatenate(
            [mlnw_ref[...], olnw_ref[...].reshape(1, D_MODEL)], axis=0)
        mixb = jnp.concatenate(
            [mlnb_ref[...], olnb_ref[...].reshape(1, D_MODEL)], axis=0)
        mixcat = jnp.concatenate([mixw, mixb], axis=1)
        wb = jnp.dot(G4, mixcat, preferred_element_type=jnp.float32)
        h = ln * wb[:, :D_MODEL] + wb[:, D_MODEL:]
        hs_ref[rows, :] = h

        @pl.when(i == 0)
        def _():
            for k in range(6):
                sm_ref[k] = 0.0

        for k, (g, xe) in enumerate(((g0, x0), (g1, x1), (g2, x2))):
            sm_ref[k] = sm_ref[k] + jnp.sum(g)
            sm_ref[3 + k] = sm_ref[3 + k] + (
                BLK - jnp.sum(xe.astype(jnp.float32)))

    @pl.when(p >= 1)
    def _():
        slot = (p - 1) % 2
        W1full = W1s_ref[slot]
        W2full = W2s_ref[slot]
        b1r = jnp.where(p == 1, b1_ref[0:1, :],
                        jnp.where(p == 2, b1_ref[1:2, :], b1_ref[2:3, :]))
        b2r = jnp.where(p == 1, b2_ref[0:1, :],
                        jnp.where(p == 2, b2_ref[1:2, :], b2_ref[2:3, :]))
        MH = BLK // 2
        for j in range(2):
            rj = pl.ds(i * BLK + j * MH, MH)
            hsj = hs_ref[rj, :]
            tj = jax.lax.dot_general(
                hsj, W1full, (((1,), (0,)), ((), ())),
                precision=jax.lax.Precision.DEFAULT,
                preferred_element_type=jnp.float32)
            gj = jax.nn.gelu(tj + b1r)
            yj = jax.lax.dot_general(
                gj, W2full, (((1,), (0,)), ((), ())),
                precision=jax.lax.Precision.DEFAULT,
                preferred_element_type=jnp.float32) + b2r
            cj = jnp.where(p == 1, c0_ref[rj, :],
                           jnp.where(p == 2, c1_ref[rj, :], c2_ref[rj, :]))

            @pl.when(p == 1)
            def _():
                acc_ref[rj, :] = hsj + cj * yj

            @pl.when(p == 2)
            def _():
                acc_ref[rj, :] = acc_ref[rj, :] + cj * yj

            @pl.when(p == NUM_EXPERTS)
            def _():
                out_ref[pl.ds(j * MH, MH), :] = acc_ref[rj, :] + cj * yj

        @pl.when((p == NUM_EXPERTS) & (i == nblk - 1))
        def _():
            aux = 0.0
            for k in range(NUM_EXPERTS):
                aux = aux + (sm_ref[3 + k] / N_TOK) * (sm_ref[k] / N_TOK)
            aux_ref[0, 0] = NUM_EXPERTS * aux


@jax.jit
def kernel(x, scout_W, scout_b, orig_ln_w, orig_ln_b, moe_ln_w, moe_ln_b,
           W1, b1, W2, b2):
    n_tok = x.shape[0]
    grid = (NUM_EXPERTS + 1, n_tok // BLK)
    out, aux = pl.pallas_call(
        _body,
        grid=grid,
        in_specs=[
            pl.BlockSpec((BLK, D_MODEL),
                         lambda p, i: (jnp.where(p == 0, i, 0), 0)),
            pl.BlockSpec((D_MODEL, NUM_EXPERTS), lambda p, i: (0, 0)),
            pl.BlockSpec(memory_space=pltpu.SMEM),
            pl.BlockSpec((D_MODEL,), lambda p, i: (0,)),
            pl.BlockSpec((D_MODEL,), lambda p, i: (0,)),
            pl.BlockSpec((NUM_EXPERTS, D_MODEL), lambda p, i: (0, 0)),
            pl.BlockSpec((NUM_EXPERTS, D_MODEL), lambda p, i: (0, 0)),
            pl.BlockSpec(memory_space=pl.ANY),
            pl.BlockSpec((NUM_EXPERTS, D_FF), lambda p, i: (0, 0)),
            pl.BlockSpec(memory_space=pl.ANY),
            pl.BlockSpec((NUM_EXPERTS, D_MODEL), lambda p, i: (0, 0)),
        ],
        out_specs=[
            pl.BlockSpec((BLK, D_MODEL),
                         lambda p, i: (jnp.where(p == NUM_EXPERTS, i, 0), 0)),
            pl.BlockSpec(memory_space=pltpu.SMEM),
        ],
        out_shape=[
            jax.ShapeDtypeStruct((n_tok, D_MODEL), jnp.float32),
            jax.ShapeDtypeStruct((1, 1), jnp.float32),
        ],
        scratch_shapes=[
            pltpu.VMEM((2, D_MODEL, D_FF), jnp.float32),
            pltpu.VMEM((2, D_FF, D_MODEL), jnp.float32),
            pltpu.VMEM((n_tok, D_MODEL), jnp.float32),
            pltpu.VMEM((n_tok, D_MODEL), jnp.float32),
            pltpu.VMEM((n_tok, 1), jnp.float32),
            pltpu.VMEM((n_tok, 1), jnp.float32),
            pltpu.VMEM((n_tok, 1), jnp.float32),
            pltpu.SMEM((8,), jnp.float32),
            pltpu.SemaphoreType.DMA,
            pltpu.SemaphoreType.DMA,
        ],
        compiler_params=pltpu.CompilerParams(
            dimension_semantics=("arbitrary", "arbitrary")),
    )(
        x, scout_W, scout_b, orig_ln_w, orig_ln_b, moe_ln_w, moe_ln_b,
        W1, b1, W2, b2,
    )
    return out, aux.reshape(())


# speedup vs baseline: 1.0347x; 1.0288x over previous
import jax
import jax.numpy as jnp
from jax.experimental import pallas as pl
from jax.experimental.pallas import tpu as pltpu

NUM_EXPERTS = 3
D_MODEL = 768
D_FF = 1536
N_TOK = 2048
BLK = 1024


def _body(x_ref, sw_ref, sb_ref, olnw_ref, olnb_ref, mlnw_ref, mlnb_ref,
          W1_hbm, b1_ref, W2_hbm, b2_ref, out_ref, aux_ref,
          W1s_ref, W2s_ref, hs_ref, acc_ref,
          c0_ref, c1_ref, c2_ref, sm_ref, sem1, sem2):
    p = pl.program_id(0)
    i = pl.program_id(1)
    nblk = pl.num_programs(1)
    rows = pl.ds(i * BLK, BLK)

    def w_copies(q, slot):
        return (pltpu.make_async_copy(W1_hbm.at[q], W1s_ref.at[slot], sem1),
                pltpu.make_async_copy(W2_hbm.at[q], W2s_ref.at[slot], sem2))

    @pl.when((i == 0) & (p < NUM_EXPERTS))
    def _():
        c1, c2 = w_copies(p, p % 2)
        c1.start()
        c2.start()

    @pl.when((i == 0) & (p >= 1))
    def _():
        q = p - 1
        slot = q % 2
        c1, c2 = w_copies(q, slot)
        c1.wait()
        c2.wait()

    @pl.when(p == 0)
    def _():
        xb = x_ref[...]
        mu = jnp.mean(xb, axis=1, keepdims=True)
        xc = xb - mu
        var = jnp.mean(xc * xc, axis=1, keepdims=True)
        ln = xc * jax.lax.rsqrt(var + 1e-6)

        lg = jnp.dot(xb, sw_ref[...], preferred_element_type=jnp.float32)
        l0 = lg[:, 0:1] + sb_ref[0]
        l1 = lg[:, 1:2] + sb_ref[1]
        l2 = lg[:, 2:3] + sb_ref[2]
        m = jnp.maximum(jnp.maximum(l0, l1), l2)
        e0 = jnp.exp(l0 - m)
        e1 = jnp.exp(l1 - m)
        e2 = jnp.exp(l2 - m)
        z = e0 + e1 + e2
        g0, g1, g2 = e0 / z, e1 / z, e2 / z

        x2 = (g2 <= g0) & (g2 <= g1)
        x1 = jnp.logical_not(x2) & (g1 <= g0)
        x0 = jnp.logical_not(x2) & jnp.logical_not(x1)
        gx = jnp.where(x0, g0, jnp.where(x1, g1, g2))
        inv = 1.0 / ((g0 + g1 + g2) - gx + 1e-6)
        c0_ref[rows, :] = jnp.where(x0, 0.0, g0 * inv)
        c1_ref[rows, :] = jnp.where(x1, 0.0, g1 * inv)
        c2_ref[rows, :] = jnp.where(x2, 0.0, g2 * inv)

        ones = jnp.ones_like(g0)
        G4 = jnp.concatenate([g0, g1, g2, ones], axis=1)
        mixw = jnp.concatenate(
            [mlnw_ref[...], olnw_ref[...].reshape(1, D_MODEL)], axis=0)
        mixb = jnp.concatenate(
            [mlnb_ref[...], olnb_ref[...].reshape(1, D_MODEL)], axis=0)
        mixcat = jnp.concatenate([mixw, mixb], axis=1)
        wb = jnp.dot(G4, mixcat, preferred_element_type=jnp.float32)
        h = ln * wb[:, :D_MODEL] + wb[:, D_MODEL:]
        hs_ref[rows, :] = h

        @pl.when(i == 0)
        def _():
            for k in range(6):
                sm_ref[k] = 0.0

        for k, (g, xe) in enumerate(((g0, x0), (g1, x1), (g2, x2))):
            sm_ref[k] = sm_ref[k] + jnp.sum(g)
            sm_ref[3 + k] = sm_ref[3 + k] + (
                BLK - jnp.sum(xe.astype(jnp.float32)))

    @pl.when(p >= 1)
    def _():
        hs = hs_ref[rows, :]
        slot = (p - 1) % 2
        b1r = jnp.where(p == 1, b1_ref[0:1, :],
                        jnp.where(p == 2, b1_ref[1:2, :], b1_ref[2:3, :]))
        t = jax.lax.dot_general(
            hs, W1s_ref[slot], (((1,), (0,)), ((), ())),
            precision=jax.lax.Precision.DEFAULT,
            preferred_element_type=jnp.float32)
        t = jax.nn.gelu(t + b1r)
        y = jax.lax.dot_general(
            t, W2s_ref[slot], (((1,), (0,)), ((), ())),
            precision=jax.lax.Precision.DEFAULT,
            preferred_element_type=jnp.float32)
        b2r = jnp.where(p == 1, b2_ref[0:1, :],
                        jnp.where(p == 2, b2_ref[1:2, :], b2_ref[2:3, :]))
        y = y + b2r
        c = jnp.where(p == 1, c0_ref[rows, :],
                      jnp.where(p == 2, c1_ref[rows, :], c2_ref[rows, :]))

        @pl.when(p == 1)
        def _():
            acc_ref[rows, :] = hs + c * y

        @pl.when(p == 2)
        def _():
            acc_ref[rows, :] = acc_ref[rows, :] + c * y

        @pl.when(p == NUM_EXPERTS)
        def _():
            out_ref[...] = acc_ref[rows, :] + c * y

            @pl.when(i == nblk - 1)
            def _():
                aux = 0.0
                for k in range(NUM_EXPERTS):
                    aux = aux + (sm_ref[3 + k] / N_TOK) * (sm_ref[k] / N_TOK)
                aux_ref[0, 0] = NUM_EXPERTS * aux


@jax.jit
def kernel(x, scout_W, scout_b, orig_ln_w, orig_ln_b, moe_ln_w, moe_ln_b,
           W1, b1, W2, b2):
    n_tok = x.shape[0]
    grid = (NUM_EXPERTS + 1, n_tok // BLK)
    out, aux = pl.pallas_call(
        _body,
        grid=grid,
        in_specs=[
            pl.BlockSpec((BLK, D_MODEL),
                         lambda p, i: (jnp.where(p == 0, i, 0), 0)),
            pl.BlockSpec((D_MODEL, NUM_EXPERTS), lambda p, i: (0, 0)),
            pl.BlockSpec(memory_space=pltpu.SMEM),
            pl.BlockSpec((D_MODEL,), lambda p, i: (0,)),
            pl.BlockSpec((D_MODEL,), lambda p, i: (0,)),
            pl.BlockSpec((NUM_EXPERTS, D_MODEL), lambda p, i: (0, 0)),
            pl.BlockSpec((NUM_EXPERTS, D_MODEL), lambda p, i: (0, 0)),
            pl.BlockSpec(memory_space=pl.ANY),
            pl.BlockSpec((NUM_EXPERTS, D_FF), lambda p, i: (0, 0)),
            pl.BlockSpec(memory_space=pl.ANY),
            pl.BlockSpec((NUM_EXPERTS, D_MODEL), lambda p, i: (0, 0)),
        ],
        out_specs=[
            pl.BlockSpec((BLK, D_MODEL),
                         lambda p, i: (jnp.where(p == NUM_EXPERTS, i, 0), 0)),
            pl.BlockSpec(memory_space=pltpu.SMEM),
        ],
        out_shape=[
            jax.ShapeDtypeStruct((n_tok, D_MODEL), jnp.float32),
            jax.ShapeDtypeStruct((1, 1), jnp.float32),
        ],
        scratch_shapes=[
            pltpu.VMEM((2, D_MODEL, D_FF), jnp.float32),
            pltpu.VMEM((2, D_FF, D_MODEL), jnp.float32),
            pltpu.VMEM((n_tok, D_MODEL), jnp.float32),
            pltpu.VMEM((n_tok, D_MODEL), jnp.float32),
            pltpu.VMEM((n_tok, 1), jnp.float32),
            pltpu.VMEM((n_tok, 1), jnp.float32),
            pltpu.VMEM((n_tok, 1), jnp.float32),
            pltpu.SMEM((8,), jnp.float32),
            pltpu.SemaphoreType.DMA,
            pltpu.SemaphoreType.DMA,
        ],
        compiler_params=pltpu.CompilerParams(
            dimension_semantics=("arbitrary", "arbitrary")),
    )(
        x, scout_W, scout_b, orig_ln_w, orig_ln_b, moe_ln_w, moe_ln_b,
        W1, b1, W2, b2,
    )
    return out, aux.reshape(())
